# baseline (speedup 1.0000x reference)
import functools

import jax
import jax.numpy as jnp
from jax import lax
from jax.experimental import pallas as pl
from jax.experimental.pallas import tpu as pltpu
from jax.experimental.pallas import tpu_sc as plsc

N = 8192
K = 16
L = 16
NW = 32
QPW = N // NW
NCHUNK = N // L

_INF = float("inf")


def _sqrt16(a):
    i = lax.bitcast_convert_type(a, jnp.int32)
    x = lax.bitcast_convert_type(
        lax.shift_right_logical(i, 1) + jnp.int32(0x1FBD1DF6), jnp.float32)
    half = jnp.float32(0.5)
    x = half * (x + a / x)
    x = half * (x + a / x)
    x = half * (x + a / x)
    return jnp.where(a > 0, x, jnp.zeros_like(x))


def _knn_body(qx_hbm, qy_hbm, qz_hbm, kx_hbm, ky_hbm, kz_hbm,
              od_hbm, ox_hbm, oy_hbm, oz_hbm,
              kxv, kyv, kzv, qxv, qyv, qzv, odv, oxv, oyv, ozv):
    c = lax.axis_index("c")
    s = lax.axis_index("s")
    wid = s * 2 + c
    base = wid * QPW

    pltpu.sync_copy(kx_hbm, kxv)
    pltpu.sync_copy(ky_hbm, kyv)
    pltpu.sync_copy(kz_hbm, kzv)
    pltpu.sync_copy(qx_hbm.at[pl.ds(base, QPW)], qxv)
    pltpu.sync_copy(qy_hbm.at[pl.ds(base, QPW)], qyv)
    pltpu.sync_copy(qz_hbm.at[pl.ds(base, QPW)], qzv)

    lane = lax.iota(jnp.int32, L)

    def do_query(qx, qy, qz, out_slot):
        def chunk(ci, carry):
            best_d, best_i, worst = carry
            off = ci * L
            dx = kxv[pl.ds(off, L)] - qx
            d = dx * dx
            dy = kyv[pl.ds(off, L)] - qy
            d = d + dy * dy
            dz = kzv[pl.ds(off, L)] - qz
            d = d + dz * dz

            def merge(args):
                d, best_d, best_i = args
                iv = lane + off
                sd, si = plsc.sort_key_val(d, iv)
                rb_d = lax.rev(best_d, (0,))
                rb_i = lax.rev(best_i, (0,))
                take = sd < rb_d
                md = jnp.where(take, sd, rb_d)
                mi = jnp.where(take, si, rb_i)
                nbd, nbi = plsc.sort_key_val(md, mi)
                return nbd, nbi, nbd[L - 1]

            def keep(args):
                _, best_d, best_i = args
                return best_d, best_i, worst

            return lax.cond(jnp.any(d < worst), merge, keep,
                            (d, best_d, best_i))

        best_d, best_i, _ = lax.fori_loop(
            0, NCHUNK, chunk,
            (jnp.full((L,), _INF, jnp.float32), jnp.zeros((L,), jnp.int32),
             jnp.float32(_INF)))

        out = pl.ds(out_slot * K, K)
        odv[out] = _sqrt16(best_d)
        oxv[out] = plsc.load_gather(kxv, [best_i])
        oyv[out] = plsc.load_gather(kyv, [best_i])
        ozv[out] = plsc.load_gather(kzv, [best_i])

    def gloop(g, _):
        qbase = g * L
        qx16 = qxv[pl.ds(qbase, L)]
        qy16 = qyv[pl.ds(qbase, L)]
        qz16 = qzv[pl.ds(qbase, L)]
        for j in range(L):
            do_query(qx16[j], qy16[j], qz16[j], qbase + j)
        return 0

    lax.fori_loop(0, QPW // L, gloop, 0)

    ob = pl.ds(base * K, QPW * K)
    pltpu.sync_copy(odv, od_hbm.at[ob])
    pltpu.sync_copy(oxv, ox_hbm.at[ob])
    pltpu.sync_copy(oyv, oy_hbm.at[ob])
    pltpu.sync_copy(ozv, oz_hbm.at[ob])


@jax.jit
def _knn_sc(qx, qy, qz, kx, ky, kz):
    flat = jax.ShapeDtypeStruct((N * K,), jnp.float32)
    f = pl.kernel(
        _knn_body,
        out_type=(flat, flat, flat, flat),
        mesh=plsc.VectorSubcoreMesh(core_axis_name="c", subcore_axis_name="s"),
        compiler_params=pltpu.CompilerParams(needs_layout_passes=False),
        scratch_types=[
            pltpu.VMEM((N,), jnp.float32),
            pltpu.VMEM((N,), jnp.float32),
            pltpu.VMEM((N,), jnp.float32),
            pltpu.VMEM((QPW,), jnp.float32),
            pltpu.VMEM((QPW,), jnp.float32),
            pltpu.VMEM((QPW,), jnp.float32),
            pltpu.VMEM((QPW * K,), jnp.float32),
            pltpu.VMEM((QPW * K,), jnp.float32),
            pltpu.VMEM((QPW * K,), jnp.float32),
            pltpu.VMEM((QPW * K,), jnp.float32),
        ],
    )
    return f(qx, qy, qz, kx, ky, kz)


def kernel(pcl_query, pcl_key):
    qx, qy, qz = (pcl_query[:, 0], pcl_query[:, 1], pcl_query[:, 2])
    kx, ky, kz = (pcl_key[:, 0], pcl_key[:, 1], pcl_key[:, 2])
    od, ox, oy, oz = _knn_sc(qx, qy, qz, kx, ky, kz)
    dists_qtk = od.reshape(N, K)
    pcl_qtk = jnp.stack(
        [ox.reshape(N, K), oy.reshape(N, K), oz.reshape(N, K)], axis=-1)
    return (pcl_qtk, dists_qtk)
